# baseline (speedup 1.0000x reference)
import functools

import jax
import jax.numpy as jnp
from jax import lax
from jax.experimental import pallas as pl
from jax.experimental.pallas import tpu as pltpu
from jax.experimental.pallas import tpu_sc as plsc

_B, _S, _D = 4, 2048, 1024
_NC, _NS = 2, 16
_NW = _NC * _NS
_SROWS = _S // _NW
_CROWS = 16
_CHE = _CROWS * _D
_NCHUNK = _SROWS // _CROWS
_XB = _S * _D


def _sc_body(x_hbm, pos_hbm, out_hbm, xbuf, pbuf):
    wid = lax.axis_index("s") * _NC + lax.axis_index("c")
    s_base = wid * _SROWS * _D

    def chunk(i, _):
        poff = s_base + i * _CHE
        pltpu.sync_copy(pos_hbm.at[pl.ds(poff, _CHE)], pbuf)

        def per_batch(b, _):
            off = b * _XB + poff
            pltpu.sync_copy(x_hbm.at[pl.ds(off, _CHE)], xbuf)

            def add16(j, _):
                o = j * 16
                xbuf[pl.ds(o, 16)] = xbuf[pl.ds(o, 16)] + pbuf[pl.ds(o, 16)]
                return 0

            lax.fori_loop(0, _CHE // 16, add16, 0)
            pltpu.sync_copy(xbuf, out_hbm.at[pl.ds(off, _CHE)])
            return 0

        lax.fori_loop(0, _B, per_batch, 0)
        return 0

    lax.fori_loop(0, _NCHUNK, chunk, 0)


@functools.partial(jax.jit)
def _sc_add(x_flat, pos_flat):
    mesh = plsc.VectorSubcoreMesh(core_axis_name="c", subcore_axis_name="s")
    return pl.kernel(
        _sc_body,
        out_type=jax.ShapeDtypeStruct((_B * _S * _D,), jnp.float32),
        mesh=mesh,
        scratch_types=[
            pltpu.VMEM((_CHE,), jnp.float32),
            pltpu.VMEM((_CHE,), jnp.float32),
        ],
    )(x_flat, pos_flat)


def kernel(x, pos_table):
    B, S, D = x.shape
    out = _sc_add(x.reshape(-1), pos_table.reshape(-1))
    return out.reshape(B, S, D)


# speedup vs baseline: 1.6620x; 1.6620x over previous
import functools

import jax
import jax.numpy as jnp
from jax import lax
from jax.experimental import pallas as pl
from jax.experimental.pallas import tpu as pltpu
from jax.experimental.pallas import tpu_sc as plsc

_B, _S, _D = 4, 2048, 1024
_NC, _NS = 2, 16
_NW = _NC * _NS
_SROWS = _S // _NW
_CROWS = 16
_CHE = _CROWS * _D
_NPOS = _SROWS // _CROWS
_NSTEP = _NPOS * _B
_XB = _S * _D


def _sc_body(x_hbm, pos_hbm, out_hbm,
             xb0, xb1, xb2, pb0, pb1,
             xs0, xs1, xs2, ps0, ps1, os0, os1, os2):
    xbufs = (xb0, xb1, xb2)
    pbufs = (pb0, pb1)
    xsems = (xs0, xs1, xs2)
    psems = (ps0, ps1)
    osems = (os0, os1, os2)

    wid = lax.axis_index("s") * _NC + lax.axis_index("c")
    s_base = wid * _SROWS * _D

    def xoff(t):
        i, b = divmod(t, _B)
        return b * _XB + s_base + i * _CHE

    def start_xload(t):
        return pltpu.async_copy(
            x_hbm.at[pl.ds(xoff(t), _CHE)], xbufs[t % 3], xsems[t % 3])

    def start_pload(i):
        return pltpu.async_copy(
            pos_hbm.at[pl.ds(s_base + i * _CHE, _CHE)],
            pbufs[i % 2], psems[i % 2])

    xloads = [None] * _NSTEP
    ploads = [None] * _NPOS
    ostores = [None] * _NSTEP

    ploads[0] = start_pload(0)
    xloads[0] = start_xload(0)
    xloads[1] = start_xload(1)

    for t in range(_NSTEP):
        i, b = divmod(t, _B)
        xb = xbufs[t % 3]
        pb = pbufs[i % 2]

        xloads[t].wait()
        if b == 0:
            ploads[i].wait()
            if i + 1 < _NPOS:
                ploads[i + 1] = start_pload(i + 1)

        @plsc.parallel_loop(0, _CHE, step=16, unroll=8)
        def _(o):
            xb[pl.ds(o, 16)] = xb[pl.ds(o, 16)] + pb[pl.ds(o, 16)]

        ostores[t] = pltpu.async_copy(
            xb, out_hbm.at[pl.ds(xoff(t), _CHE)], osems[t % 3])

        if t + 2 < _NSTEP:
            if t - 1 >= 0:
                ostores[t - 1].wait()
            xloads[t + 2] = start_xload(t + 2)

    ostores[_NSTEP - 3].wait()
    ostores[_NSTEP - 2].wait()
    ostores[_NSTEP - 1].wait()


@functools.partial(jax.jit)
def _sc_add(x_flat, pos_flat):
    mesh = plsc.VectorSubcoreMesh(core_axis_name="c", subcore_axis_name="s")
    return pl.kernel(
        _sc_body,
        out_type=jax.ShapeDtypeStruct((_B * _S * _D,), jnp.float32),
        mesh=mesh,
        scratch_types=[
            pltpu.VMEM((_CHE,), jnp.float32),
            pltpu.VMEM((_CHE,), jnp.float32),
            pltpu.VMEM((_CHE,), jnp.float32),
            pltpu.VMEM((_CHE,), jnp.float32),
            pltpu.VMEM((_CHE,), jnp.float32),
            pltpu.SemaphoreType.DMA,
            pltpu.SemaphoreType.DMA,
            pltpu.SemaphoreType.DMA,
            pltpu.SemaphoreType.DMA,
            pltpu.SemaphoreType.DMA,
            pltpu.SemaphoreType.DMA,
            pltpu.SemaphoreType.DMA,
            pltpu.SemaphoreType.DMA,
        ],
    )(x_flat, pos_flat)


def kernel(x, pos_table):
    B, S, D = x.shape
    out = _sc_add(x.reshape(-1), pos_table.reshape(-1))
    return out.reshape(B, S, D)


# speedup vs baseline: 3.4769x; 2.0921x over previous
import functools

import jax
import jax.numpy as jnp
from jax import lax
from jax.experimental import pallas as pl
from jax.experimental.pallas import tpu as pltpu
from jax.experimental.pallas import tpu_sc as plsc

_B, _S, _D = 4, 2048, 1024
_NC, _NS = 2, 16
_NW = _NC * _NS
_SROWS = _S // _NW
_CROWS = 16
_NPOS = _SROWS // _CROWS
_NSTEP = _NPOS * _B


def _sc_body(x_hbm, pos_hbm, out_hbm,
             xb0, xb1, xb2, pb0, pb1,
             xs0, xs1, xs2, ps0, ps1, os0, os1, os2):
    xbufs = (xb0, xb1, xb2)
    pbufs = (pb0, pb1)
    xsems = (xs0, xs1, xs2)
    psems = (ps0, ps1)
    osems = (os0, os1, os2)

    wid = lax.axis_index("s") * _NC + lax.axis_index("c")
    row_base = wid * _SROWS

    def rows(t):
        i, b = divmod(t, _B)
        return b, row_base + i * _CROWS

    def start_xload(t):
        b, r0 = rows(t)
        return pltpu.async_copy(
            x_hbm.at[b, pl.ds(r0, _CROWS)], xbufs[t % 3], xsems[t % 3])

    def start_pload(i):
        return pltpu.async_copy(
            pos_hbm.at[pl.ds(row_base + i * _CROWS, _CROWS)],
            pbufs[i % 2], psems[i % 2])

    xloads = [None] * _NSTEP
    ploads = [None] * _NPOS
    ostores = [None] * _NSTEP

    ploads[0] = start_pload(0)
    xloads[0] = start_xload(0)
    xloads[1] = start_xload(1)

    for t in range(_NSTEP):
        i, b = divmod(t, _B)
        xb = xbufs[t % 3]
        pb = pbufs[i % 2]

        xloads[t].wait()
        if b == 0:
            ploads[i].wait()
            if i + 1 < _NPOS:
                ploads[i + 1] = start_pload(i + 1)

        @plsc.parallel_loop(0, _D, step=16, unroll=2)
        def _(o):
            for r in range(_CROWS):
                xb[r, pl.ds(o, 16)] = xb[r, pl.ds(o, 16)] + pb[r, pl.ds(o, 16)]

        bb, r0 = rows(t)
        ostores[t] = pltpu.async_copy(
            xb, out_hbm.at[bb, pl.ds(r0, _CROWS)], osems[t % 3])

        if t + 2 < _NSTEP:
            if t - 1 >= 0:
                ostores[t - 1].wait()
            xloads[t + 2] = start_xload(t + 2)

    ostores[_NSTEP - 3].wait()
    ostores[_NSTEP - 2].wait()
    ostores[_NSTEP - 1].wait()


@functools.partial(jax.jit)
def _sc_add(x, pos_table):
    mesh = plsc.VectorSubcoreMesh(core_axis_name="c", subcore_axis_name="s")
    return pl.kernel(
        _sc_body,
        out_type=jax.ShapeDtypeStruct((_B, _S, _D), jnp.float32),
        mesh=mesh,
        scratch_types=[
            pltpu.VMEM((_CROWS, _D), jnp.float32),
            pltpu.VMEM((_CROWS, _D), jnp.float32),
            pltpu.VMEM((_CROWS, _D), jnp.float32),
            pltpu.VMEM((_CROWS, _D), jnp.float32),
            pltpu.VMEM((_CROWS, _D), jnp.float32),
            pltpu.SemaphoreType.DMA,
            pltpu.SemaphoreType.DMA,
            pltpu.SemaphoreType.DMA,
            pltpu.SemaphoreType.DMA,
            pltpu.SemaphoreType.DMA,
            pltpu.SemaphoreType.DMA,
            pltpu.SemaphoreType.DMA,
            pltpu.SemaphoreType.DMA,
        ],
    )(x, pos_table)


def kernel(x, pos_table):
    return _sc_add(x, pos_table)


# speedup vs baseline: 3.5636x; 1.0249x over previous
import functools

import jax
import jax.numpy as jnp
from jax import lax
from jax.experimental import pallas as pl
from jax.experimental.pallas import tpu as pltpu
from jax.experimental.pallas import tpu_sc as plsc

_B, _S, _D = 4, 2048, 1024
_NC, _NS = 2, 16
_NW = _NC * _NS
_SROWS = _S // _NW
_CROWS = 16
_NPOS = _SROWS // _CROWS
_NSTEP = _NPOS * _B
_NXB = 3
_NOB = 2


def _sc_body(x_hbm, pos_hbm, out_hbm,
             xb0, xb1, xb2, ob0, ob1, pb0, pb1,
             xs0, xs1, xs2, os0, os1, ps0, ps1):
    xbufs = (xb0, xb1, xb2)
    obufs = (ob0, ob1)
    pbufs = (pb0, pb1)
    xsems = (xs0, xs1, xs2)
    osems = (os0, os1)
    psems = (ps0, ps1)

    wid = lax.axis_index("s") * _NC + lax.axis_index("c")
    row_base = wid * _SROWS

    def rows(t):
        i, b = divmod(t, _B)
        return b, row_base + i * _CROWS

    def start_xload(t):
        b, r0 = rows(t)
        return pltpu.async_copy(
            x_hbm.at[b, pl.ds(r0, _CROWS)], xbufs[t % _NXB], xsems[t % _NXB])

    def start_pload(i):
        return pltpu.async_copy(
            pos_hbm.at[pl.ds(row_base + i * _CROWS, _CROWS)],
            pbufs[i % 2], psems[i % 2])

    xloads = [None] * _NSTEP
    ploads = [None] * _NPOS
    ostores = [None] * _NSTEP

    ploads[0] = start_pload(0)
    ploads[1] = start_pload(1)
    for t in range(_NXB):
        xloads[t] = start_xload(t)

    for t in range(_NSTEP):
        i, b = divmod(t, _B)
        xb = xbufs[t % _NXB]
        ob = obufs[t % _NOB]
        pb = pbufs[i % 2]

        xloads[t].wait()
        if b == 0:
            ploads[i].wait()
        if t - _NOB >= 0:
            ostores[t - _NOB].wait()

        @plsc.parallel_loop(0, _D, step=16, unroll=2)
        def _(o):
            for r in range(_CROWS):
                ob[r, pl.ds(o, 16)] = xb[r, pl.ds(o, 16)] + pb[r, pl.ds(o, 16)]

        bb, r0 = rows(t)
        ostores[t] = pltpu.async_copy(
            ob, out_hbm.at[bb, pl.ds(r0, _CROWS)], osems[t % _NOB])
        if t + _NXB < _NSTEP:
            xloads[t + _NXB] = start_xload(t + _NXB)
        if b == _B - 1 and i + 2 < _NPOS:
            ploads[i + 2] = start_pload(i + 2)

    ostores[_NSTEP - 2].wait()
    ostores[_NSTEP - 1].wait()


@functools.partial(jax.jit)
def _sc_add(x, pos_table):
    mesh = plsc.VectorSubcoreMesh(core_axis_name="c", subcore_axis_name="s")
    return pl.kernel(
        _sc_body,
        out_type=jax.ShapeDtypeStruct((_B, _S, _D), jnp.float32),
        mesh=mesh,
        compiler_params=pltpu.CompilerParams(
            disable_bounds_checks=True,
            disable_semaphore_checks=True,
        ),
        scratch_types=[
            pltpu.VMEM((_CROWS, _D), jnp.float32),
            pltpu.VMEM((_CROWS, _D), jnp.float32),
            pltpu.VMEM((_CROWS, _D), jnp.float32),
            pltpu.VMEM((_CROWS, _D), jnp.float32),
            pltpu.VMEM((_CROWS, _D), jnp.float32),
            pltpu.VMEM((_CROWS, _D), jnp.float32),
            pltpu.VMEM((_CROWS, _D), jnp.float32),
            pltpu.SemaphoreType.DMA,
            pltpu.SemaphoreType.DMA,
            pltpu.SemaphoreType.DMA,
            pltpu.SemaphoreType.DMA,
            pltpu.SemaphoreType.DMA,
            pltpu.SemaphoreType.DMA,
            pltpu.SemaphoreType.DMA,
        ],
    )(x, pos_table)


def kernel(x, pos_table):
    return _sc_add(x, pos_table)


# speedup vs baseline: 3.7728x; 1.0587x over previous
import functools

import jax
import jax.numpy as jnp
from jax import lax
from jax.experimental import pallas as pl
from jax.experimental.pallas import tpu as pltpu
from jax.experimental.pallas import tpu_sc as plsc

_B, _S, _D = 4, 2048, 1024
_NC, _NS = 2, 16
_NW = _NC * _NS
_SROWS = _S // _NW
_CROWS = 16
_NPOS = _SROWS // _CROWS
_NSTEP = _NPOS * _B
_NXB = 5


def _sc_body(x_hbm, pos_hbm, out_hbm,
             xb0, xb1, xb2, xb3, xb4, pb0, pb1,
             xs0, xs1, xs2, xs3, xs4, os0, os1, os2, os3, os4, ps0, ps1):
    xbufs = (xb0, xb1, xb2, xb3, xb4)
    pbufs = (pb0, pb1)
    xsems = (xs0, xs1, xs2, xs3, xs4)
    osems = (os0, os1, os2, os3, os4)
    psems = (ps0, ps1)

    wid = lax.axis_index("s") * _NC + lax.axis_index("c")
    row_base = wid * _SROWS

    def rows(t):
        i, b = divmod(t, _B)
        return b, row_base + i * _CROWS

    def start_xload(t):
        b, r0 = rows(t)
        return pltpu.async_copy(
            x_hbm.at[b, pl.ds(r0, _CROWS)], xbufs[t % _NXB], xsems[t % _NXB])

    def start_pload(i):
        return pltpu.async_copy(
            pos_hbm.at[pl.ds(row_base + i * _CROWS, _CROWS)],
            pbufs[i % 2], psems[i % 2])

    xloads = [None] * _NSTEP
    ploads = [None] * _NPOS
    ostores = [None] * _NSTEP

    ploads[0] = start_pload(0)
    ploads[1] = start_pload(1)
    for t in range(min(_NXB - 1, _NSTEP)):
        xloads[t] = start_xload(t)

    for t in range(_NSTEP):
        i, b = divmod(t, _B)
        xb = xbufs[t % _NXB]
        pb = pbufs[i % 2]

        xloads[t].wait()
        if b == 0:
            ploads[i].wait()

        @plsc.parallel_loop(0, _D, step=16, unroll=2)
        def _(o):
            for r in range(_CROWS):
                plsc.addupdate(xb.at[r, pl.ds(o, 16)], pb[r, pl.ds(o, 16)])

        bb, r0 = rows(t)
        ostores[t] = pltpu.async_copy(
            xb, out_hbm.at[bb, pl.ds(r0, _CROWS)], osems[t % _NXB])
        if t + _NXB - 1 < _NSTEP:
            if t - 1 >= 0:
                ostores[t - 1].wait()
            xloads[t + _NXB - 1] = start_xload(t + _NXB - 1)
        if b == _B - 1 and i + 2 < _NPOS:
            ploads[i + 2] = start_pload(i + 2)

    for t in range(_NSTEP - _NXB, _NSTEP):
        ostores[t].wait()


@functools.partial(jax.jit)
def _sc_add(x, pos_table):
    mesh = plsc.VectorSubcoreMesh(core_axis_name="c", subcore_axis_name="s")
    return pl.kernel(
        _sc_body,
        out_type=jax.ShapeDtypeStruct((_B, _S, _D), jnp.float32),
        mesh=mesh,
        compiler_params=pltpu.CompilerParams(
            disable_bounds_checks=True,
            disable_semaphore_checks=True,
        ),
        scratch_types=[
            pltpu.VMEM((_CROWS, _D), jnp.float32),
            pltpu.VMEM((_CROWS, _D), jnp.float32),
            pltpu.VMEM((_CROWS, _D), jnp.float32),
            pltpu.VMEM((_CROWS, _D), jnp.float32),
            pltpu.VMEM((_CROWS, _D), jnp.float32),
            pltpu.VMEM((_CROWS, _D), jnp.float32),
            pltpu.VMEM((_CROWS, _D), jnp.float32),
            pltpu.SemaphoreType.DMA,
            pltpu.SemaphoreType.DMA,
            pltpu.SemaphoreType.DMA,
            pltpu.SemaphoreType.DMA,
            pltpu.SemaphoreType.DMA,
            pltpu.SemaphoreType.DMA,
            pltpu.SemaphoreType.DMA,
            pltpu.SemaphoreType.DMA,
            pltpu.SemaphoreType.DMA,
            pltpu.SemaphoreType.DMA,
            pltpu.SemaphoreType.DMA,
            pltpu.SemaphoreType.DMA,
        ],
    )(x, pos_table)


def kernel(x, pos_table):
    return _sc_add(x, pos_table)


# speedup vs baseline: 4.2871x; 1.1363x over previous
import functools

import jax
import jax.numpy as jnp
from jax import lax
from jax.experimental import pallas as pl
from jax.experimental.pallas import tpu as pltpu
from jax.experimental.pallas import tpu_sc as plsc

_B, _S, _D = 4, 2048, 1024
_NC, _NS = 2, 16
_NW = _NC * _NS
_SROWS = _S // _NW
_CROWS = 16
_NPOS = _SROWS // _CROWS
_NSTEP = _NPOS * _B
_NXB = 5


def _sc_body(x_hbm, pos_hbm, out_hbm,
             xb0, xb1, xb2, xb3, xb4, pb0, pb1,
             xs0, xs1, xs2, xs3, xs4, os0, os1, os2, os3, os4, ps0, ps1):
    xbufs = (xb0, xb1, xb2, xb3, xb4)
    pbufs = (pb0, pb1)
    xsems = (xs0, xs1, xs2, xs3, xs4)
    osems = (os0, os1, os2, os3, os4)
    psems = (ps0, ps1)

    wid = lax.axis_index("s") * _NC + lax.axis_index("c")
    row_base = wid * _SROWS

    def rows(t):
        i, b = divmod(t, _B)
        return b, row_base + i * _CROWS

    def start_xload(t):
        b, r0 = rows(t)
        return pltpu.async_copy(
            x_hbm.at[b, pl.ds(r0, _CROWS)], xbufs[t % _NXB], xsems[t % _NXB])

    def start_pload(i):
        return pltpu.async_copy(
            pos_hbm.at[pl.ds(row_base + i * _CROWS, _CROWS)],
            pbufs[i % 2], psems[i % 2])

    xloads = [None] * _NSTEP
    ploads = [None] * _NPOS
    ostores = [None] * _NSTEP

    ploads[0] = start_pload(0)
    ploads[1] = start_pload(1)
    for t in range(min(_NXB - 1, _NSTEP)):
        xloads[t] = start_xload(t)

    for t in range(_NSTEP):
        i, b = divmod(t, _B)
        xb = xbufs[t % _NXB]
        pb = pbufs[i % 2]

        xloads[t].wait()
        if b == 0:
            ploads[i].wait()

        @plsc.parallel_loop(0, _CROWS * _D, step=16, unroll=4)
        def _(o):
            r = o // _D
            c = o % _D
            plsc.addupdate(xb.at[r, pl.ds(c, 16)], pb[r, pl.ds(c, 16)])

        bb, r0 = rows(t)
        ostores[t] = pltpu.async_copy(
            xb, out_hbm.at[bb, pl.ds(r0, _CROWS)], osems[t % _NXB])
        if t + _NXB - 1 < _NSTEP:
            if t - 1 >= 0:
                ostores[t - 1].wait()
            xloads[t + _NXB - 1] = start_xload(t + _NXB - 1)
        if b == _B - 1 and i + 2 < _NPOS:
            ploads[i + 2] = start_pload(i + 2)

    for t in range(_NSTEP - _NXB, _NSTEP):
        ostores[t].wait()


@functools.partial(jax.jit)
def _sc_add(x, pos_table):
    mesh = plsc.VectorSubcoreMesh(core_axis_name="c", subcore_axis_name="s")
    return pl.kernel(
        _sc_body,
        out_type=jax.ShapeDtypeStruct((_B, _S, _D), jnp.float32),
        mesh=mesh,
        compiler_params=pltpu.CompilerParams(
            disable_bounds_checks=True,
            disable_semaphore_checks=True,
            skip_device_barrier=True,
        ),
        scratch_types=[
            pltpu.VMEM((_CROWS, _D), jnp.float32),
            pltpu.VMEM((_CROWS, _D), jnp.float32),
            pltpu.VMEM((_CROWS, _D), jnp.float32),
            pltpu.VMEM((_CROWS, _D), jnp.float32),
            pltpu.VMEM((_CROWS, _D), jnp.float32),
            pltpu.VMEM((_CROWS, _D), jnp.float32),
            pltpu.VMEM((_CROWS, _D), jnp.float32),
            pltpu.SemaphoreType.DMA,
            pltpu.SemaphoreType.DMA,
            pltpu.SemaphoreType.DMA,
            pltpu.SemaphoreType.DMA,
            pltpu.SemaphoreType.DMA,
            pltpu.SemaphoreType.DMA,
            pltpu.SemaphoreType.DMA,
            pltpu.SemaphoreType.DMA,
            pltpu.SemaphoreType.DMA,
            pltpu.SemaphoreType.DMA,
            pltpu.SemaphoreType.DMA,
            pltpu.SemaphoreType.DMA,
        ],
    )(x, pos_table)


def kernel(x, pos_table):
    return _sc_add(x, pos_table)
